# baseline (speedup 1.0000x reference)
import functools

import jax
import jax.numpy as jnp
from jax import lax
from jax.experimental import pallas as pl
from jax.experimental.pallas import tpu as pltpu
from jax.experimental.pallas import tpu_sc as plsc

_EPS = 1e-3
_K = 16
_KH = 8

_SC_CORES = 2
_SC_SUBCORES = 16
_SC_WORKERS = _SC_CORES * _SC_SUBCORES
_SC_CHUNK = 128


def _knn_body(new_ref, srct_ref, out_ref, *, set_id, nbatch, npts):
    b = pl.program_id(0)
    new = new_ref[0]
    srct = srct_ref[0]
    d2 = None
    for d in range(3):
        diff = new[:, d:d + 1] - srct[d:d + 1, :]
        sq = diff * diff
        d2 = sq if d2 is None else d2 + sq
    colsf = lax.broadcasted_iota(jnp.int32, d2.shape, 1).astype(jnp.float32)
    base = (set_id * nbatch + b) * npts
    bigf = jnp.float32(3e38)
    for j in range(_KH):
        m = jnp.min(d2, axis=1, keepdims=True)
        cand = jnp.where(d2 == m, colsf, bigf)
        idxj = jnp.min(cand, axis=1, keepdims=True)
        out_ref[0, :, j:j + 1] = idxj.astype(jnp.int32) + base
        d2 = jnp.where(colsf == idxj, jnp.float32(jnp.inf), d2)


def _knn_topk(new_pts, srct_s, set_id, rows_per_tile=256):
    nbatch, npts, _ = new_pts.shape
    ntiles = npts // rows_per_tile
    return pl.pallas_call(
        functools.partial(_knn_body, set_id=set_id, nbatch=nbatch, npts=npts),
        grid=(nbatch, ntiles),
        in_specs=[
            pl.BlockSpec((1, rows_per_tile, 3), lambda b, t: (b, t, 0)),
            pl.BlockSpec((1, 3, npts), lambda b, t: (b, 0, 0)),
        ],
        out_specs=pl.BlockSpec((1, rows_per_tile, _KH),
                               lambda b, t: (b, t, 0)),
        out_shape=jax.ShapeDtypeStruct((nbatch, npts, _KH), jnp.int32),
    )(new_pts, srct_s)


def _sc_gather(table, idx):
    nidx = idx.shape[0]
    dim = table.shape[1]
    per_w = nidx // _SC_WORKERS
    nchunks = per_w // _SC_CHUNK
    mesh = plsc.VectorSubcoreMesh(core_axis_name="c", subcore_axis_name="s")

    @functools.partial(
        pl.kernel,
        mesh=mesh,
        out_type=jax.ShapeDtypeStruct((nidx, dim), jnp.float32),
        scratch_types=[
            pltpu.VMEM((_SC_CHUNK,), jnp.int32),
            pltpu.VMEM((_SC_CHUNK, dim), jnp.float32),
            pltpu.SemaphoreType.DMA,
        ],
    )
    def gather_k(table_hbm, idx_hbm, out_hbm, idx_v, rows_v, sem):
        wid = lax.axis_index("s") * _SC_CORES + lax.axis_index("c")
        base = wid * per_w

        def body(ci, carry):
            off = base + ci * _SC_CHUNK
            pltpu.sync_copy(idx_hbm.at[pl.ds(off, _SC_CHUNK)], idx_v)
            pltpu.async_copy(table_hbm.at[idx_v], rows_v, sem).wait()
            pltpu.sync_copy(rows_v, out_hbm.at[pl.ds(off, _SC_CHUNK)])
            return carry

        lax.fori_loop(0, nchunks, body, 0)

    return gather_k(table, idx)


def _p1_body(g_ref, new_ref, w1t_ref, b1_ref, y_ref, s_ref):
    t = pl.program_id(0)
    g = g_ref[...]
    rows = g.shape[0]
    nv = new_ref[...]
    nrep = jnp.broadcast_to(nv[:, None, :], (rows // _KH, _KH, 3)).reshape(rows, 3)
    resi = g[:, 0:3] - nrep
    dist = jnp.sqrt(jnp.sum(resi * resi, axis=1, keepdims=True))
    h0 = jnp.concatenate([resi, dist], axis=1)
    y = jnp.dot(h0, w1t_ref[...],
                preferred_element_type=jnp.float32) + b1_ref[...]
    y_ref[...] = y

    @pl.when(t == 0)
    def _():
        s_ref[...] = jnp.zeros_like(s_ref)

    s_ref[0:1, :] += jnp.sum(y, axis=0, keepdims=True)
    s_ref[1:2, :] += jnp.sum(y * y, axis=0, keepdims=True)


def _mid_body(y_ref, st_ref, wt_ref, b_ref, g_ref, be_ref, out_ref, s_ref,
              *, inv_s, write_h):
    t = pl.program_id(0)
    mean = st_ref[0:1, :] * inv_s
    var = st_ref[1:2, :] * inv_s - mean * mean
    scale = g_ref[...] * lax.rsqrt(var + _EPS)
    h = jnp.maximum((y_ref[...] - mean) * scale + be_ref[...], 0.0)
    y_next = jnp.dot(h, wt_ref[...], preferred_element_type=jnp.float32) + b_ref[...]
    out_ref[...] = h if write_h else y_next

    @pl.when(t == 0)
    def _():
        s_ref[...] = jnp.zeros_like(s_ref)

    s_ref[0:1, :] += jnp.sum(y_next, axis=0, keepdims=True)
    s_ref[1:2, :] += jnp.sum(y_next * y_next, axis=0, keepdims=True)


def _p3lite_body(y_ref, st_ref, g_ref, be_ref, m1_ref, m2_ref, *, inv_s):
    t = pl.program_id(0)
    mean = st_ref[0:1, :] * inv_s
    var = st_ref[1:2, :] * inv_s - mean * mean
    scale = g_ref[...] * lax.rsqrt(var + _EPS)
    h = jnp.maximum((y_ref[...] - mean) * scale + be_ref[...], 0.0)

    @pl.when(t == 0)
    def _():
        m1_ref[...] = jnp.zeros_like(m1_ref)
        m2_ref[...] = jnp.zeros_like(m2_ref)

    m1_ref[...] += jnp.sum(h, axis=0, keepdims=True)
    m2_ref[...] += lax.dot_general(h, h, (((0,), (0,)), ((), ())),
                                   preferred_element_type=jnp.float32)


def _p4_body(ya_ref, yb_ref, ga_ref, gb_ref, st2_ref, m1_ref, m2_ref,
             wt_ref, b_ref, gam_ref, be_ref, g2_ref, be2_ref, o_ref,
             *, inv_s, rows):
    mean2 = st2_ref[0:1, :] * inv_s
    var2 = st2_ref[1:2, :] * inv_s - mean2 * mean2
    scale2 = g2_ref[...] * lax.rsqrt(var2 + _EPS)

    wt = wt_ref[...]
    bv = b_ref[...]
    hp = lax.Precision.HIGHEST
    ym = jnp.dot(m1_ref[...], wt, preferred_element_type=jnp.float32,
                 precision=hp)
    t1 = jnp.dot(m2_ref[...], wt, preferred_element_type=jnp.float32,
                 precision=hp)
    q3 = (jnp.sum(wt * t1, axis=0, keepdims=True)
          + 2.0 * bv * ym + (1.0 / inv_s) * bv * bv)
    mean3 = (ym + (1.0 / inv_s) * bv) * inv_s
    var3 = q3 * inv_s - mean3 * mean3
    scale3 = gam_ref[...] * lax.rsqrt(var3 + _EPS)
    npoint = rows // _KH

    def half(y_ref, g_ref):
        h = jnp.maximum((y_ref[...] - mean2) * scale2 + be2_ref[...], 0.0)
        y3 = jnp.dot(h, wt, preferred_element_type=jnp.float32) + bv
        x3 = jnp.maximum((y3 - mean3) * scale3 + be_ref[...], 0.0)
        e = jnp.exp(jnp.max(x3, axis=1, keepdims=True))
        f = g_ref[:, 0:67]
        num = jnp.sum((e * f).reshape(npoint, _KH, 67), axis=1)
        den = jnp.sum(e.reshape(npoint, _KH, 1), axis=1)
        return num, den

    na, da = half(ya_ref, ga_ref)
    nb, db = half(yb_ref, gb_ref)
    o_ref[...] = (na + nb) / (da + db)


def _mlp_fusion(gath_a, gath_b, new_pts2d, params):
    sh = gath_a.shape[0]
    s_total = 2 * sh
    rows = 4096
    ntiles = sh // rows
    inv_s = 1.0 / s_total
    (w1, b1, g1, be1), (w2, b2, g2, be2), (w3, b3, g3, be3) = params
    c1, c2, c3 = w1.shape[0], w2.shape[0], w3.shape[0]

    def stat_spec(c):
        return pl.BlockSpec((2, c), lambda t: (0, 0))

    def full(shp):
        return pl.BlockSpec(shp, lambda t: (0, 0))

    def vec(a):
        return a.reshape(1, -1)

    def p1(gath_h):
        return pl.pallas_call(
            _p1_body,
            grid=(ntiles,),
            in_specs=[
                pl.BlockSpec((rows, gath_h.shape[1]), lambda t: (t, 0)),
                pl.BlockSpec((rows // _KH, 3), lambda t: (t, 0)),
                full((4, c1)), full((1, c1)),
            ],
            out_specs=[pl.BlockSpec((rows, c1), lambda t: (t, 0)), stat_spec(c1)],
            out_shape=[jax.ShapeDtypeStruct((sh, c1), jnp.float32),
                       jax.ShapeDtypeStruct((2, c1), jnp.float32)],
        )(gath_h, new_pts2d, w1.T, vec(b1))

    def mid(y_h, st, wt, b, g, be, cin, cout, write_h):
        return pl.pallas_call(
            functools.partial(_mid_body, inv_s=inv_s, write_h=write_h),
            grid=(ntiles,),
            in_specs=[
                pl.BlockSpec((rows, cin), lambda t: (t, 0)),
                stat_spec(cin), full((cin, cout)), full((1, cout)),
                full((1, cin)), full((1, cin)),
            ],
            out_specs=[pl.BlockSpec((rows, cin if write_h else cout),
                                    lambda t: (t, 0)), stat_spec(cout)],
            out_shape=[jax.ShapeDtypeStruct((sh, cin if write_h else cout),
                                            jnp.float32),
                       jax.ShapeDtypeStruct((2, cout), jnp.float32)],
        )(y_h, st, wt, b, g, be)

    y1a, s1a = p1(gath_a)
    y1b, s1b = p1(gath_b)
    s1 = s1a + s1b

    y2a, s2a = mid(y1a, s1, w2.T, vec(b2), vec(g1), vec(be1), c1, c2, False)
    y2b, s2b = mid(y1b, s1, w2.T, vec(b2), vec(g1), vec(be1), c1, c2, False)
    s2 = s2a + s2b

    def p3lite(y_h):
        return pl.pallas_call(
            functools.partial(_p3lite_body, inv_s=inv_s),
            grid=(ntiles,),
            in_specs=[pl.BlockSpec((rows, c2), lambda t: (t, 0)),
                      stat_spec(c2), full((1, c2)), full((1, c2))],
            out_specs=[pl.BlockSpec((1, c2), lambda t: (0, 0)),
                       pl.BlockSpec((c2, c2), lambda t: (0, 0))],
            out_shape=[jax.ShapeDtypeStruct((1, c2), jnp.float32),
                       jax.ShapeDtypeStruct((c2, c2), jnp.float32)],
        )(y_h, s2, vec(g2), vec(be2))

    m1a, m2a = p3lite(y2a)
    m1b, m2b = p3lite(y2b)
    m1 = m1a + m1b
    m2 = m2a + m2b

    rows4 = 2048
    nt4 = sh // rows4
    out = pl.pallas_call(
        functools.partial(_p4_body, inv_s=inv_s, rows=rows4),
        grid=(nt4,),
        in_specs=[
            pl.BlockSpec((rows4, c2), lambda t: (t, 0)),
            pl.BlockSpec((rows4, c2), lambda t: (t, 0)),
            pl.BlockSpec((rows4, gath_a.shape[1]), lambda t: (t, 0)),
            pl.BlockSpec((rows4, gath_b.shape[1]), lambda t: (t, 0)),
            stat_spec(c2), full((1, c2)), full((c2, c2)),
            full((c2, c3)), full((1, c3)),
            full((1, c3)), full((1, c3)),
            full((1, c2)), full((1, c2)),
        ],
        out_specs=pl.BlockSpec((rows4 // _KH, 67), lambda t: (t, 0)),
        out_shape=jax.ShapeDtypeStruct((sh // _KH, 67), jnp.float32),
    )(y2a, y2b, gath_a, gath_b, s2, m1, m2,
      w3.T, vec(b3), vec(g3), vec(be3), vec(g2), vec(be2))
    return out


def kernel(points1, points2, features1, features2, k, t, params):
    nbatch, npts, _ = points1.shape
    nfeat = features1.shape[1]
    n2 = npts // 2
    n1 = npts - n2

    with jax.ensure_compile_time_eval():
        perm_key = jax.random.key(42)
        perm1, perm2 = [], []
        for i in range(nbatch):
            ka = jax.random.fold_in(perm_key, 2 * i)
            kb = jax.random.fold_in(perm_key, 2 * i + 1)
            perm1.append(jax.random.permutation(ka, npts)[:n1])
            perm2.append(jax.random.permutation(kb, npts)[:n2])
    new_rows = [
        jnp.concatenate([points1[i][perm1[i]], points2[i][perm2[i]]], axis=0)
        for i in range(nbatch)
    ]
    new_pts = jnp.stack(new_rows, axis=0)

    src = jnp.stack([points1, points2], axis=0)
    srct = jnp.transpose(src, (0, 1, 3, 2))

    feats = jnp.stack([features1, features2], axis=0)
    featst = jnp.transpose(feats, (0, 1, 3, 2))
    dim = 3 + nfeat
    pad = (-dim) % 128
    table = jnp.concatenate(
        [src, featst, jnp.zeros((2, nbatch, npts, pad), jnp.float32)], axis=-1)
    table = table.reshape(2 * nbatch * npts, dim + pad)

    idxa = _knn_topk(new_pts, srct[0], 0).reshape(-1)
    gath_a = _sc_gather(table, idxa)
    idxb = _knn_topk(new_pts, srct[1], 1).reshape(-1)
    gath_b = _sc_gather(table, idxb)

    out = _mlp_fusion(gath_a, gath_b,
                      new_pts.reshape(nbatch * npts, 3), params)
    return jnp.transpose(out.reshape(nbatch, npts, dim), (0, 2, 1))


# speedup vs baseline: 1.0039x; 1.0039x over previous
import functools

import jax
import jax.numpy as jnp
from jax import lax
from jax.experimental import pallas as pl
from jax.experimental.pallas import tpu as pltpu
from jax.experimental.pallas import tpu_sc as plsc

_EPS = 1e-3
_K = 16
_KH = 8

_SC_CORES = 2
_SC_SUBCORES = 16
_SC_WORKERS = _SC_CORES * _SC_SUBCORES
_SC_CHUNK = 128


def _knn_body(new_ref, srct_ref, out_ref, *, set_id, nbatch, npts):
    b = pl.program_id(0)
    new = new_ref[0]
    srct = srct_ref[0]
    d2 = None
    for d in range(3):
        diff = new[:, d:d + 1] - srct[d:d + 1, :]
        sq = diff * diff
        d2 = sq if d2 is None else d2 + sq
    colsf = lax.broadcasted_iota(jnp.int32, d2.shape, 1).astype(jnp.float32)
    base = (set_id * nbatch + b) * npts
    bigf = jnp.float32(3e38)
    for j in range(_KH):
        m = jnp.min(d2, axis=1, keepdims=True)
        cand = jnp.where(d2 == m, colsf, bigf)
        idxj = jnp.min(cand, axis=1, keepdims=True)
        out_ref[0, :, j:j + 1] = idxj.astype(jnp.int32) + base
        d2 = jnp.where(colsf == idxj, jnp.float32(jnp.inf), d2)


def _knn_topk(new_pts, srct_s, set_id, rows_per_tile=256):
    nbatch, npts, _ = new_pts.shape
    ntiles = npts // rows_per_tile
    return pl.pallas_call(
        functools.partial(_knn_body, set_id=set_id, nbatch=nbatch, npts=npts),
        grid=(nbatch, ntiles),
        in_specs=[
            pl.BlockSpec((1, rows_per_tile, 3), lambda b, t: (b, t, 0)),
            pl.BlockSpec((1, 3, npts), lambda b, t: (b, 0, 0)),
        ],
        out_specs=pl.BlockSpec((1, rows_per_tile, _KH),
                               lambda b, t: (b, t, 0)),
        out_shape=jax.ShapeDtypeStruct((nbatch, npts, _KH), jnp.int32),
    )(new_pts, srct_s)


def _sc_gather(table, idx):
    nidx = idx.shape[0]
    dim = table.shape[1]
    per_w = nidx // _SC_WORKERS
    nchunks = per_w // _SC_CHUNK
    mesh = plsc.VectorSubcoreMesh(core_axis_name="c", subcore_axis_name="s")

    @functools.partial(
        pl.kernel,
        mesh=mesh,
        out_type=jax.ShapeDtypeStruct((nidx, dim), jnp.float32),
        scratch_types=[
            pltpu.VMEM((_SC_CHUNK,), jnp.int32),
            pltpu.VMEM((_SC_CHUNK, dim), jnp.float32),
            pltpu.SemaphoreType.DMA,
        ],
    )
    def gather_k(table_hbm, idx_hbm, out_hbm, idx_v, rows_v, sem):
        wid = lax.axis_index("s") * _SC_CORES + lax.axis_index("c")
        base = wid * per_w

        def body(ci, carry):
            off = base + ci * _SC_CHUNK
            pltpu.sync_copy(idx_hbm.at[pl.ds(off, _SC_CHUNK)], idx_v)
            pltpu.async_copy(table_hbm.at[idx_v], rows_v, sem).wait()
            pltpu.sync_copy(rows_v, out_hbm.at[pl.ds(off, _SC_CHUNK)])
            return carry

        lax.fori_loop(0, nchunks, body, 0)

    return gather_k(table, idx)


def _p1_body(g_ref, new_ref, w1t_ref, b1_ref, y_ref, s_ref):
    t = pl.program_id(0)
    g = g_ref[...]
    rows = g.shape[0]
    nv = new_ref[...]
    nrep = jnp.broadcast_to(nv[:, None, :], (rows // _KH, _KH, 3)).reshape(rows, 3)
    resi = g[:, 0:3] - nrep
    dist = jnp.sqrt(jnp.sum(resi * resi, axis=1, keepdims=True))
    h0 = jnp.concatenate([resi, dist], axis=1)
    y = jnp.dot(h0, w1t_ref[...],
                preferred_element_type=jnp.float32) + b1_ref[...]
    y_ref[...] = y

    @pl.when(t == 0)
    def _():
        s_ref[...] = jnp.zeros_like(s_ref)

    s_ref[0:1, :] += jnp.sum(y, axis=0, keepdims=True)
    s_ref[1:2, :] += jnp.sum(y * y, axis=0, keepdims=True)


def _mid_body(y_ref, st_ref, wt_ref, b_ref, g_ref, be_ref, out_ref, s_ref,
              *, inv_s, write_h):
    t = pl.program_id(0)
    mean = st_ref[0:1, :] * inv_s
    var = st_ref[1:2, :] * inv_s - mean * mean
    scale = g_ref[...] * lax.rsqrt(var + _EPS)
    h = jnp.maximum((y_ref[...] - mean) * scale + be_ref[...], 0.0)
    y_next = jnp.dot(h, wt_ref[...], preferred_element_type=jnp.float32) + b_ref[...]
    out_ref[...] = h if write_h else y_next

    @pl.when(t == 0)
    def _():
        s_ref[...] = jnp.zeros_like(s_ref)

    s_ref[0:1, :] += jnp.sum(y_next, axis=0, keepdims=True)
    s_ref[1:2, :] += jnp.sum(y_next * y_next, axis=0, keepdims=True)


def _p4_body(ha_ref, hb_ref, ga_ref, gb_ref, st_ref, wt_ref, b_ref,
             gam_ref, be_ref, o_ref, *, inv_s, rows):
    mean = st_ref[0:1, :] * inv_s
    var = st_ref[1:2, :] * inv_s - mean * mean
    scale = gam_ref[...] * lax.rsqrt(var + _EPS)
    npoint = rows // _KH

    def half(h_ref, g_ref):
        y3 = jnp.dot(h_ref[...], wt_ref[...],
                     preferred_element_type=jnp.float32) + b_ref[...]
        x3 = jnp.maximum((y3 - mean) * scale + be_ref[...], 0.0)
        e = jnp.exp(jnp.max(x3, axis=1, keepdims=True))
        f = g_ref[:, 0:67]
        num = jnp.sum((e * f).reshape(npoint, _KH, 67), axis=1)
        den = jnp.sum(e.reshape(npoint, _KH, 1), axis=1)
        return num, den

    na, da = half(ha_ref, ga_ref)
    nb, db = half(hb_ref, gb_ref)
    o_ref[...] = (na + nb) / (da + db)


def _mlp_fusion(gath_a, gath_b, new_pts2d, params):
    sh = gath_a.shape[0]
    s_total = 2 * sh
    rows = 4096
    ntiles = sh // rows
    inv_s = 1.0 / s_total
    (w1, b1, g1, be1), (w2, b2, g2, be2), (w3, b3, g3, be3) = params
    c1, c2, c3 = w1.shape[0], w2.shape[0], w3.shape[0]

    def stat_spec(c):
        return pl.BlockSpec((2, c), lambda t: (0, 0))

    def full(shp):
        return pl.BlockSpec(shp, lambda t: (0, 0))

    def vec(a):
        return a.reshape(1, -1)

    def p1(gath_h):
        return pl.pallas_call(
            _p1_body,
            grid=(ntiles,),
            in_specs=[
                pl.BlockSpec((rows, gath_h.shape[1]), lambda t: (t, 0)),
                pl.BlockSpec((rows // _KH, 3), lambda t: (t, 0)),
                full((4, c1)), full((1, c1)),
            ],
            out_specs=[pl.BlockSpec((rows, c1), lambda t: (t, 0)), stat_spec(c1)],
            out_shape=[jax.ShapeDtypeStruct((sh, c1), jnp.float32),
                       jax.ShapeDtypeStruct((2, c1), jnp.float32)],
        )(gath_h, new_pts2d, w1.T, vec(b1))

    def mid(y_h, st, wt, b, g, be, cin, cout, write_h):
        return pl.pallas_call(
            functools.partial(_mid_body, inv_s=inv_s, write_h=write_h),
            grid=(ntiles,),
            in_specs=[
                pl.BlockSpec((rows, cin), lambda t: (t, 0)),
                stat_spec(cin), full((cin, cout)), full((1, cout)),
                full((1, cin)), full((1, cin)),
            ],
            out_specs=[pl.BlockSpec((rows, cin if write_h else cout),
                                    lambda t: (t, 0)), stat_spec(cout)],
            out_shape=[jax.ShapeDtypeStruct((sh, cin if write_h else cout),
                                            jnp.float32),
                       jax.ShapeDtypeStruct((2, cout), jnp.float32)],
        )(y_h, st, wt, b, g, be)

    y1a, s1a = p1(gath_a)
    y1b, s1b = p1(gath_b)
    s1 = s1a + s1b

    y2a, s2a = mid(y1a, s1, w2.T, vec(b2), vec(g1), vec(be1), c1, c2, False)
    y2b, s2b = mid(y1b, s1, w2.T, vec(b2), vec(g1), vec(be1), c1, c2, False)
    s2 = s2a + s2b

    h2a, s3a = mid(y2a, s2, w3.T, vec(b3), vec(g2), vec(be2), c2, c3, True)
    h2b, s3b = mid(y2b, s2, w3.T, vec(b3), vec(g2), vec(be2), c2, c3, True)
    s3 = s3a + s3b

    rows4 = 2048
    nt4 = sh // rows4
    out = pl.pallas_call(
        functools.partial(_p4_body, inv_s=inv_s, rows=rows4),
        grid=(nt4,),
        in_specs=[
            pl.BlockSpec((rows4, c2), lambda t: (t, 0)),
            pl.BlockSpec((rows4, c2), lambda t: (t, 0)),
            pl.BlockSpec((rows4, gath_a.shape[1]), lambda t: (t, 0)),
            pl.BlockSpec((rows4, gath_b.shape[1]), lambda t: (t, 0)),
            stat_spec(c3), full((c2, c3)), full((1, c3)),
            full((1, c3)), full((1, c3)),
        ],
        out_specs=pl.BlockSpec((rows4 // _KH, 67), lambda t: (t, 0)),
        out_shape=jax.ShapeDtypeStruct((sh // _KH, 67), jnp.float32),
    )(h2a, h2b, gath_a, gath_b, s3, w3.T, vec(b3), vec(g3), vec(be3))
    return out


def kernel(points1, points2, features1, features2, k, t, params):
    nbatch, npts, _ = points1.shape
    nfeat = features1.shape[1]
    n2 = npts // 2
    n1 = npts - n2

    with jax.ensure_compile_time_eval():
        perm_key = jax.random.key(42)
        perm1, perm2 = [], []
        for i in range(nbatch):
            ka = jax.random.fold_in(perm_key, 2 * i)
            kb = jax.random.fold_in(perm_key, 2 * i + 1)
            perm1.append(jax.random.permutation(ka, npts)[:n1])
            perm2.append(jax.random.permutation(kb, npts)[:n2])
    new_rows = [
        jnp.concatenate([points1[i][perm1[i]], points2[i][perm2[i]]], axis=0)
        for i in range(nbatch)
    ]
    new_pts = jnp.stack(new_rows, axis=0)

    src = jnp.stack([points1, points2], axis=0)
    srct = jnp.transpose(src, (0, 1, 3, 2))

    feats = jnp.stack([features1, features2], axis=0)
    featst = jnp.transpose(feats, (0, 1, 3, 2))
    dim = 3 + nfeat
    pad = (-dim) % 128
    table = jnp.concatenate(
        [src, featst, jnp.zeros((2, nbatch, npts, pad), jnp.float32)], axis=-1)
    table = table.reshape(2 * nbatch * npts, dim + pad)

    idxa = _knn_topk(new_pts, srct[0], 0).reshape(-1)
    gath_a = _sc_gather(table, idxa)
    idxb = _knn_topk(new_pts, srct[1], 1).reshape(-1)
    gath_b = _sc_gather(table, idxb)

    out = _mlp_fusion(gath_a, gath_b,
                      new_pts.reshape(nbatch * npts, 3), params)
    return jnp.transpose(out.reshape(nbatch, npts, dim), (0, 2, 1))
